# Initial kernel scaffold; baseline (speedup 1.0000x reference)
import jax
import jax.numpy as jnp
from jax.experimental import pallas as pl


def kernel(x, edge_index, Ws, bs, gammas, betas, fcW, fcb):
    raise NotImplementedError("write your pallas kernel here")


# speedup vs baseline: 7.3615x; 7.3615x over previous
import dataclasses
import functools

import jax
import jax.numpy as jnp
from jax import lax
from jax.experimental import pallas as pl
from jax.experimental.pallas import tpu as pltpu
from jax.experimental.pallas import tpu_sc as plsc

N = 10000
E = 320000
D = 128
L = 5

NC = 2
NS = 16
RPT = 624
RTAIL = N - NS * RPT
EPC = E // NC
EPT = EPC // NS
CH = 80
NCH = EPT // CH

_mesh = plsc.VectorSubcoreMesh(core_axis_name="c", subcore_axis_name="s")

_cp = pltpu.CompilerParams()
if "needs_layout_passes" in pltpu.CompilerParams.__dataclass_fields__:
    _cp = dataclasses.replace(_cp, needs_layout_passes=False)


@functools.partial(
    pl.kernel,
    out_type=jax.ShapeDtypeStruct((NC * N, D), jnp.float32),
    mesh=_mesh,
    scratch_types=[
        pltpu.VMEM((CH,), jnp.int32),
        pltpu.VMEM((CH,), jnp.int32),
        pltpu.VMEM((CH, D), jnp.float32),
        pltpu.VMEM_SHARED((N, D), jnp.float32),
        pltpu.SemaphoreType.DMA,
    ],
)
def _agg_sc(y_hbm, src_hbm, dst_hbm, out_hbm, sidx, didx, rows, acc_sh, sem):
    cid = lax.axis_index("c")
    sid = lax.axis_index("s")
    rbase = sid * RPT

    pltpu.sync_copy(y_hbm.at[pl.ds(rbase, RPT)], acc_sh.at[pl.ds(rbase, RPT)])

    @pl.when(sid == NS - 1)
    def _():
        pltpu.sync_copy(y_hbm.at[pl.ds(NS * RPT, RTAIL)],
                        acc_sh.at[pl.ds(NS * RPT, RTAIL)])

    plsc.subcore_barrier()

    ebase = cid * EPC + sid * EPT

    @pl.loop(0, NCH)
    def _(j):
        pltpu.sync_copy(src_hbm.at[pl.ds(ebase + j * CH, CH)], sidx)
        pltpu.sync_copy(dst_hbm.at[pl.ds(ebase + j * CH, CH)], didx)
        pltpu.async_copy(y_hbm.at[sidx], rows, sem).wait()
        pltpu.sync_copy(rows, acc_sh.at[didx], add=True)

    plsc.subcore_barrier()
    pltpu.sync_copy(acc_sh.at[pl.ds(rbase, RPT)],
                    out_hbm.at[pl.ds(cid * N + rbase, RPT)])

    @pl.when(sid == NS - 1)
    def _():
        pltpu.sync_copy(acc_sh.at[pl.ds(NS * RPT, RTAIL)],
                        out_hbm.at[pl.ds(cid * N + NS * RPT, RTAIL)])


@functools.partial(
    pl.kernel,
    out_type=jax.ShapeDtypeStruct((E,), jnp.float32),
    mesh=_mesh,
    scratch_types=[
        pltpu.VMEM((CH,), jnp.int32),
        pltpu.VMEM((CH,), jnp.int32),
        pltpu.VMEM((CH, D), jnp.float32),
        pltpu.VMEM((CH, D), jnp.float32),
        pltpu.VMEM((CH, 16), jnp.float32),
        pltpu.VMEM((CH,), jnp.float32),
        pltpu.VMEM((16,), jnp.float32),
        pltpu.SemaphoreType.DMA,
        pltpu.SemaphoreType.DMA,
    ],
    compiler_params=_cp,
)
def _edge_sc(g_hbm, h_hbm, src_hbm, dst_hbm, fcb_hbm, out_hbm,
             sidx, didx, gbuf, hbuf, accb, obuf, fcbv, sem1, sem2):
    cid = lax.axis_index("c")
    sid = lax.axis_index("s")
    pltpu.sync_copy(fcb_hbm, fcbv)
    ebase = cid * EPC + sid * EPT

    @pl.loop(0, NCH)
    def _(j):
        eoff = ebase + j * CH
        pltpu.sync_copy(src_hbm.at[pl.ds(eoff, CH)], sidx)
        pltpu.sync_copy(dst_hbm.at[pl.ds(eoff, CH)], didx)
        cg = pltpu.async_copy(g_hbm.at[sidx], gbuf, sem1)
        ch = pltpu.async_copy(h_hbm.at[didx], hbuf, sem2)
        cg.wait()
        ch.wait()

        @pl.loop(0, CH)
        def _(r):
            acc = gbuf[r, pl.ds(0, 16)] * hbuf[r, pl.ds(0, 16)]
            for t in range(1, D // 16):
                acc = acc + gbuf[r, pl.ds(16 * t, 16)] * hbuf[r, pl.ds(16 * t, 16)]
            accb[r] = acc

        lanes = lax.iota(jnp.int32, 16)

        @pl.loop(0, CH // 16)
        def _(q):
            rows = q * 16 + lanes
            s = plsc.load_gather(accb, [rows, jnp.full((16,), 0, jnp.int32)])
            for c in range(1, 16):
                s = s + plsc.load_gather(accb, [rows, jnp.full((16,), c, jnp.int32)])
            v = s + fcbv[...]
            obuf[pl.ds(q * 16, 16)] = 1.0 / (1.0 + jnp.exp(-v))

        pltpu.sync_copy(obuf, out_hbm.at[pl.ds(eoff, CH)])


def _tc_pre(degp, x, w0):
    def body(degp_ref, x_ref, w_ref, y_ref, dis_ref):
        deg = degp_ref[0:N, 0:1] + degp_ref[N:2 * N, 0:1] - 1.0
        dis = lax.rsqrt(deg)
        xw = jnp.dot(x_ref[...], w_ref[...],
                     preferred_element_type=jnp.float32,
                     precision=lax.Precision.HIGHEST)
        y_ref[...] = xw * dis
        dis_ref[...] = dis

    return pl.pallas_call(
        body,
        out_shape=[jax.ShapeDtypeStruct((N, D), jnp.float32),
                   jax.ShapeDtypeStruct((N, 1), jnp.float32)],
    )(degp, x, w0)


def _bn_relu(accp_ref, y_ref, dis_ref, b_ref, ga_ref, be_ref):
    t = dis_ref[...] * (accp_ref[0:N] + accp_ref[N:2 * N] - y_ref[...]) + b_ref[...]
    mu = jnp.mean(t, axis=0, keepdims=True)
    var = jnp.mean((t - mu) ** 2, axis=0, keepdims=True)
    return jnp.maximum(
        ga_ref[...] * (t - mu) * lax.rsqrt(var + 1e-5) + be_ref[...], 0.0)


def _tc_mid(accp, y, dis, b, ga, be, wn):
    def body(accp_ref, y_ref, dis_ref, b_ref, ga_ref, be_ref, w_ref, yn_ref):
        hn = _bn_relu(accp_ref, y_ref, dis_ref, b_ref, ga_ref, be_ref)
        yn_ref[...] = jnp.dot(hn, w_ref[...],
                              preferred_element_type=jnp.float32,
                              precision=lax.Precision.HIGHEST) * dis_ref[...]

    return pl.pallas_call(
        body,
        out_shape=jax.ShapeDtypeStruct((N, D), jnp.float32),
    )(accp, y, dis, b, ga, be, wn)


def _tc_last(accp, y, dis, b, ga, be, fcw_row):
    def body(accp_ref, y_ref, dis_ref, b_ref, ga_ref, be_ref, fcw_ref,
             g_ref, h_ref):
        hn = _bn_relu(accp_ref, y_ref, dis_ref, b_ref, ga_ref, be_ref)
        h_ref[...] = hn
        g_ref[...] = hn * fcw_ref[...]

    return pl.pallas_call(
        body,
        out_shape=[jax.ShapeDtypeStruct((N, D), jnp.float32),
                   jax.ShapeDtypeStruct((N, D), jnp.float32)],
    )(accp, y, dis, b, ga, be, fcw_row)


def kernel(x, edge_index, Ws, bs, gammas, betas, fcW, fcb):
    src = edge_index[0]
    dst = edge_index[1]

    degp = _agg_sc(jnp.ones((N, D), jnp.float32), dst, dst)
    y, dis = _tc_pre(degp, x, Ws[0])
    for i in range(L):
        accp = _agg_sc(y, src, dst)
        if i < L - 1:
            y = _tc_mid(accp, y, dis, bs[i][None], gammas[i][None],
                        betas[i][None], Ws[i + 1])
        else:
            g, h5 = _tc_last(accp, y, dis, bs[i][None], gammas[i][None],
                             betas[i][None], fcW[:, 0][None])
    out = _edge_sc(g, h5, src, dst, jnp.broadcast_to(fcb, (16,)))
    return out[:, None]


# speedup vs baseline: 18.5325x; 2.5175x over previous
import dataclasses
import functools

import jax
import jax.numpy as jnp
from jax import lax
from jax.experimental import pallas as pl
from jax.experimental.pallas import tpu as pltpu
from jax.experimental.pallas import tpu_sc as plsc

N = 10000
E = 320000
D = 128
L = 5

NC = 2
NS = 16
RPT = 624
RTAIL = N - NS * RPT
EPC = E // NC
EPT = EPC // NS
CH = 80
NCH = EPT // CH
PD = 5
NG = NCH // PD
CHA = 40
NCHA = EPT // CHA
NGA = NCHA // PD

_mesh = plsc.VectorSubcoreMesh(core_axis_name="c", subcore_axis_name="s")

_cp = pltpu.CompilerParams()
if "needs_layout_passes" in pltpu.CompilerParams.__dataclass_fields__:
    _cp = dataclasses.replace(_cp, needs_layout_passes=False)


def _init_acc(src_hbm, acc_sh, sid):
    rbase = sid * RPT
    pltpu.sync_copy(src_hbm.at[pl.ds(rbase, RPT)], acc_sh.at[pl.ds(rbase, RPT)])

    @pl.when(sid == NS - 1)
    def _():
        pltpu.sync_copy(src_hbm.at[pl.ds(NS * RPT, RTAIL)],
                        acc_sh.at[pl.ds(NS * RPT, RTAIL)])


def _write_acc(acc_sh, out_hbm, cid, sid):
    rbase = sid * RPT
    pltpu.sync_copy(acc_sh.at[pl.ds(rbase, RPT)],
                    out_hbm.at[pl.ds(cid * N + rbase, RPT)])

    @pl.when(sid == NS - 1)
    def _():
        pltpu.sync_copy(acc_sh.at[pl.ds(NS * RPT, RTAIL)],
                        out_hbm.at[pl.ds(cid * N + NS * RPT, RTAIL)])


@functools.partial(
    pl.kernel,
    out_type=jax.ShapeDtypeStruct((NC * N, D), jnp.float32),
    mesh=_mesh,
    scratch_types=(
        [pltpu.VMEM((EPT,), jnp.int32)]
        + [pltpu.VMEM((CHA,), jnp.int32)] * PD
        + [pltpu.VMEM((CHA, D), jnp.float32)] * PD
        + [pltpu.VMEM_SHARED((N, D), jnp.float32)]
        + [pltpu.SemaphoreType.DMA] * (3 * PD)
    ),
)
def _agg_sc(y_hbm, src_hbm, dst_hbm, out_hbm, sbuf, *rest):
    didx = rest[0:PD]
    rows = rest[PD:2 * PD]
    acc_sh = rest[2 * PD]
    isem = rest[2 * PD + 1:3 * PD + 1]
    gsem = rest[3 * PD + 1:4 * PD + 1]
    ssem = rest[4 * PD + 1:5 * PD + 1]

    cid = lax.axis_index("c")
    sid = lax.axis_index("s")
    ebase = cid * EPC + sid * EPT

    pltpu.sync_copy(src_hbm.at[pl.ds(ebase, EPT)], sbuf)
    _init_acc(y_hbm, acc_sh, sid)
    plsc.subcore_barrier()

    def fire(j, b):
        pltpu.async_copy(dst_hbm.at[pl.ds(ebase + j * CHA, CHA)], didx[b],
                         isem[b])
        pltpu.async_copy(y_hbm.at[sbuf.at[pl.ds(j * CHA, CHA)]], rows[b],
                         gsem[b])

    def scat(b):
        pltpu.make_async_copy(y_hbm.at[sbuf.at[pl.ds(0, CHA)]], rows[b],
                              gsem[b]).wait()
        pltpu.make_async_copy(dst_hbm.at[pl.ds(ebase, CHA)], didx[b],
                              isem[b]).wait()
        pltpu.async_copy(rows[b], acc_sh.at[didx[b]], ssem[b], add=True)

    def scat_wait(b):
        pltpu.make_async_copy(rows[b], acc_sh.at[didx[b]], ssem[b]).wait()

    for b in range(PD):
        fire(b, b)
    for b in range(PD):
        scat(b)

    @pl.loop(1, NGA)
    def _(g):
        base = g * PD
        for b in range(PD):
            scat_wait(b)
            fire(base + b, b)
        for b in range(PD):
            scat(b)

    for b in range(PD):
        scat_wait(b)

    plsc.subcore_barrier()
    _write_acc(acc_sh, out_hbm, cid, sid)


@functools.partial(
    pl.kernel,
    out_type=jax.ShapeDtypeStruct((NC * N, D), jnp.float32),
    mesh=_mesh,
    scratch_types=(
        [pltpu.VMEM((CH, D), jnp.float32)]
        + [pltpu.VMEM((CH,), jnp.int32)] * PD
        + [pltpu.VMEM_SHARED((N, D), jnp.float32)]
        + [pltpu.SemaphoreType.DMA] * (2 * PD)
    ),
)
def _deg_sc(ones_hbm, dst_hbm, out_hbm, ones_v, *rest):
    didx = rest[0:PD]
    acc_sh = rest[PD]
    isem = rest[PD + 1:2 * PD + 1]
    ssem = rest[2 * PD + 1:3 * PD + 1]

    cid = lax.axis_index("c")
    sid = lax.axis_index("s")
    ebase = cid * EPC + sid * EPT

    pltpu.sync_copy(ones_hbm.at[pl.ds(0, CH)], ones_v)
    _init_acc(ones_hbm, acc_sh, sid)
    plsc.subcore_barrier()

    def fire(j, b):
        pltpu.async_copy(dst_hbm.at[pl.ds(ebase + j * CH, CH)], didx[b],
                         isem[b])

    def scat(b):
        pltpu.make_async_copy(dst_hbm.at[pl.ds(ebase, CH)], didx[b],
                              isem[b]).wait()
        pltpu.async_copy(ones_v, acc_sh.at[didx[b]], ssem[b], add=True)

    def scat_wait(b):
        pltpu.make_async_copy(ones_v, acc_sh.at[didx[b]], ssem[b]).wait()

    for b in range(PD):
        fire(b, b)
    for b in range(PD):
        scat(b)

    @pl.loop(1, NG)
    def _(g):
        base = g * PD
        for b in range(PD):
            scat_wait(b)
            fire(base + b, b)
        for b in range(PD):
            scat(b)

    for b in range(PD):
        scat_wait(b)

    plsc.subcore_barrier()
    _write_acc(acc_sh, out_hbm, cid, sid)


@functools.partial(
    pl.kernel,
    out_type=jax.ShapeDtypeStruct((E,), jnp.float32),
    mesh=_mesh,
    scratch_types=(
        [pltpu.VMEM((EPT,), jnp.int32)] * 2
        + [pltpu.VMEM((CH, D), jnp.float32)] * 4
        + [pltpu.VMEM((CH, 16), jnp.float32)]
        + [pltpu.VMEM((CH,), jnp.float32)] * 2
        + [pltpu.VMEM((16,), jnp.float32)]
        + [pltpu.SemaphoreType.DMA] * 4
    ),
    compiler_params=_cp,
)
def _edge_sc(g_hbm, h_hbm, src_hbm, dst_hbm, fcb_hbm, out_hbm,
             sbuf, dbuf, gb0, hb0, gb1, hb1, accb, ob0, ob1, fcbv,
             gs0, gs1, os0, os1):
    cid = lax.axis_index("c")
    sid = lax.axis_index("s")
    ebase = cid * EPC + sid * EPT

    pltpu.sync_copy(fcb_hbm, fcbv)
    pltpu.sync_copy(src_hbm.at[pl.ds(ebase, EPT)], sbuf)
    pltpu.sync_copy(dst_hbm.at[pl.ds(ebase, EPT)], dbuf)

    gbuf = (gb0, gb1)
    hbuf = (hb0, hb1)
    obuf = (ob0, ob1)
    gsem = (gs0, gs1)
    osem = (os0, os1)
    lanes = lax.iota(jnp.int32, 16)

    def fire(j, b):
        pltpu.async_copy(g_hbm.at[sbuf.at[pl.ds(j * CH, CH)]], gbuf[b],
                         gsem[b])
        pltpu.async_copy(h_hbm.at[dbuf.at[pl.ds(j * CH, CH)]], hbuf[b],
                         gsem[b])

    def gwait(b):
        pltpu.make_async_copy(g_hbm.at[sbuf.at[pl.ds(0, CH)]], gbuf[b],
                              gsem[b]).wait()
        pltpu.make_async_copy(h_hbm.at[dbuf.at[pl.ds(0, CH)]], hbuf[b],
                              gsem[b]).wait()

    def owait(b):
        pltpu.make_async_copy(obuf[b], out_hbm.at[pl.ds(ebase, CH)],
                              osem[b]).wait()

    def compute(j, b):
        gb, hb, ob = gbuf[b], hbuf[b], obuf[b]

        @pl.loop(0, CH)
        def _(r):
            acc = gb[r, pl.ds(0, 16)] * hb[r, pl.ds(0, 16)]
            for t in range(1, D // 16):
                acc = acc + gb[r, pl.ds(16 * t, 16)] * hb[r, pl.ds(16 * t, 16)]
            accb[r] = acc

        @pl.loop(0, CH // 16)
        def _(q):
            rows16 = q * 16 + lanes
            s = plsc.load_gather(accb, [rows16, jnp.full((16,), 0, jnp.int32)])
            for c in range(1, 16):
                s = s + plsc.load_gather(accb,
                                         [rows16, jnp.full((16,), c, jnp.int32)])
            v = s + fcbv[...]
            ob[pl.ds(q * 16, 16)] = 1.0 / (1.0 + jnp.exp(-v))

        pltpu.async_copy(ob, out_hbm.at[pl.ds(ebase + j * CH, CH)], osem[b])

    fire(0, 0)
    fire(1, 1)
    gwait(0)
    compute(0, 0)
    fire(2, 0)
    gwait(1)
    compute(1, 1)
    fire(3, 1)

    @pl.loop(1, (NCH - 3) // 2)
    def _(t):
        for b in range(2):
            j = 2 * t + b
            gwait(b)
            owait(b)
            compute(j, b)
            fire(j + 2, b)

    for b in range(2):
        gwait(b)
        owait(b)
        compute(NCH - 3 + b, b)
    fire(NCH - 1, 0)
    gwait(0)
    owait(0)
    compute(NCH - 1, 0)
    owait(0)
    owait(1)


def _tc_pre(degp, x, w0):
    def body(degp_ref, x_ref, w_ref, y_ref, dis_ref):
        deg = degp_ref[0:N, 0:1] + degp_ref[N:2 * N, 0:1] - 1.0
        dis = lax.rsqrt(deg)
        xw = jnp.dot(x_ref[...], w_ref[...],
                     preferred_element_type=jnp.float32,
                     precision=lax.Precision.HIGHEST)
        y_ref[...] = xw * dis
        dis_ref[...] = dis

    return pl.pallas_call(
        body,
        out_shape=[jax.ShapeDtypeStruct((N, D), jnp.float32),
                   jax.ShapeDtypeStruct((N, 1), jnp.float32)],
    )(degp, x, w0)


def _bn_relu(accp_ref, y_ref, dis_ref, b_ref, ga_ref, be_ref):
    t = dis_ref[...] * (accp_ref[0:N] + accp_ref[N:2 * N] - y_ref[...]) + b_ref[...]
    mu = jnp.mean(t, axis=0, keepdims=True)
    var = jnp.mean((t - mu) ** 2, axis=0, keepdims=True)
    return jnp.maximum(
        ga_ref[...] * (t - mu) * lax.rsqrt(var + 1e-5) + be_ref[...], 0.0)


def _tc_mid(accp, y, dis, b, ga, be, wn):
    def body(accp_ref, y_ref, dis_ref, b_ref, ga_ref, be_ref, w_ref, yn_ref):
        hn = _bn_relu(accp_ref, y_ref, dis_ref, b_ref, ga_ref, be_ref)
        yn_ref[...] = jnp.dot(hn, w_ref[...],
                              preferred_element_type=jnp.float32,
                              precision=lax.Precision.HIGHEST) * dis_ref[...]

    return pl.pallas_call(
        body,
        out_shape=jax.ShapeDtypeStruct((N, D), jnp.float32),
    )(accp, y, dis, b, ga, be, wn)


def _tc_last(accp, y, dis, b, ga, be, fcw_row):
    def body(accp_ref, y_ref, dis_ref, b_ref, ga_ref, be_ref, fcw_ref,
             g_ref, h_ref):
        hn = _bn_relu(accp_ref, y_ref, dis_ref, b_ref, ga_ref, be_ref)
        h_ref[...] = hn
        g_ref[...] = hn * fcw_ref[...]

    return pl.pallas_call(
        body,
        out_shape=[jax.ShapeDtypeStruct((N, D), jnp.float32),
                   jax.ShapeDtypeStruct((N, D), jnp.float32)],
    )(accp, y, dis, b, ga, be, fcw_row)


def kernel(x, edge_index, Ws, bs, gammas, betas, fcW, fcb):
    src = edge_index[0]
    dst = edge_index[1]

    degp = _deg_sc(jnp.ones((N, D), jnp.float32), dst)
    y, dis = _tc_pre(degp, x, Ws[0])
    for i in range(L):
        accp = _agg_sc(y, src, dst)
        if i < L - 1:
            y = _tc_mid(accp, y, dis, bs[i][None], gammas[i][None],
                        betas[i][None], Ws[i + 1])
        else:
            g, h5 = _tc_last(accp, y, dis, bs[i][None], gammas[i][None],
                             betas[i][None], fcW[:, 0][None])
    out = _edge_sc(g, h5, src, dst, jnp.broadcast_to(fcb, (16,)))
    return out[:, None]


# speedup vs baseline: 18.7337x; 1.0109x over previous
import dataclasses
import functools

import jax
import jax.numpy as jnp
from jax import lax
from jax.experimental import pallas as pl
from jax.experimental.pallas import tpu as pltpu
from jax.experimental.pallas import tpu_sc as plsc

N = 10000
E = 320000
D = 128
L = 5

NC = 2
NS = 16
RPT = 624
RTAIL = N - NS * RPT
EPC = E // NC
EPT = EPC // NS
CH = 80
NCH = EPT // CH
PD = 5
NG = NCH // PD
CHA = 40
NCHA = EPT // CHA
NGA = NCHA // PD

_mesh = plsc.VectorSubcoreMesh(core_axis_name="c", subcore_axis_name="s")

_cp = pltpu.CompilerParams()
if "needs_layout_passes" in pltpu.CompilerParams.__dataclass_fields__:
    _cp = dataclasses.replace(_cp, needs_layout_passes=False)


def _init_acc(src_hbm, acc_sh, sid):
    rbase = sid * RPT
    pltpu.sync_copy(src_hbm.at[pl.ds(rbase, RPT)], acc_sh.at[pl.ds(rbase, RPT)])

    @pl.when(sid == NS - 1)
    def _():
        pltpu.sync_copy(src_hbm.at[pl.ds(NS * RPT, RTAIL)],
                        acc_sh.at[pl.ds(NS * RPT, RTAIL)])


def _write_acc(acc_sh, out_hbm, cid, sid):
    rbase = sid * RPT
    pltpu.sync_copy(acc_sh.at[pl.ds(rbase, RPT)],
                    out_hbm.at[pl.ds(cid * N + rbase, RPT)])

    @pl.when(sid == NS - 1)
    def _():
        pltpu.sync_copy(acc_sh.at[pl.ds(NS * RPT, RTAIL)],
                        out_hbm.at[pl.ds(cid * N + NS * RPT, RTAIL)])


@functools.partial(
    pl.kernel,
    out_type=jax.ShapeDtypeStruct((NC * N, D), jnp.float32),
    mesh=_mesh,
    scratch_types=(
        [pltpu.VMEM((EPT,), jnp.int32)]
        + [pltpu.VMEM((CHA,), jnp.int32)] * PD
        + [pltpu.VMEM((CHA, D), jnp.float32)] * PD
        + [pltpu.VMEM_SHARED((N, D), jnp.float32)]
        + [pltpu.SemaphoreType.DMA] * (3 * PD)
    ),
)
def _agg_sc(y_hbm, src_hbm, dst_hbm, out_hbm, sbuf, *rest):
    didx = rest[0:PD]
    rows = rest[PD:2 * PD]
    acc_sh = rest[2 * PD]
    isem = rest[2 * PD + 1:3 * PD + 1]
    gsem = rest[3 * PD + 1:4 * PD + 1]
    ssem = rest[4 * PD + 1:5 * PD + 1]

    cid = lax.axis_index("c")
    sid = lax.axis_index("s")
    ebase = cid * EPC + sid * EPT

    pltpu.sync_copy(src_hbm.at[pl.ds(ebase, EPT)], sbuf)
    _init_acc(y_hbm, acc_sh, sid)
    plsc.subcore_barrier()

    def fire(j, b):
        pltpu.async_copy(dst_hbm.at[pl.ds(ebase + j * CHA, CHA)], didx[b],
                         isem[b])
        pltpu.async_copy(y_hbm.at[sbuf.at[pl.ds(j * CHA, CHA)]], rows[b],
                         gsem[b])

    def scat(b):
        pltpu.make_async_copy(y_hbm.at[sbuf.at[pl.ds(0, CHA)]], rows[b],
                              gsem[b]).wait()
        pltpu.make_async_copy(dst_hbm.at[pl.ds(ebase, CHA)], didx[b],
                              isem[b]).wait()
        pltpu.async_copy(rows[b], acc_sh.at[didx[b]], ssem[b], add=True)

    def scat_wait(b):
        pltpu.make_async_copy(rows[b], acc_sh.at[didx[b]], ssem[b]).wait()

    for b in range(PD):
        fire(b, b)
    for b in range(PD):
        scat(b)

    @pl.loop(1, NGA)
    def _(g):
        base = g * PD
        for b in range(PD):
            scat_wait(b)
            fire(base + b, b)
        for b in range(PD):
            scat(b)

    for b in range(PD):
        scat_wait(b)

    plsc.subcore_barrier()
    _write_acc(acc_sh, out_hbm, cid, sid)


@functools.partial(
    pl.kernel,
    out_type=jax.ShapeDtypeStruct((NC * N, D), jnp.float32),
    mesh=_mesh,
    scratch_types=(
        [pltpu.VMEM((CH, D), jnp.float32)]
        + [pltpu.VMEM((CH,), jnp.int32)] * PD
        + [pltpu.VMEM_SHARED((N, D), jnp.float32)]
        + [pltpu.SemaphoreType.DMA] * (2 * PD)
    ),
)
def _deg_sc(ones_hbm, dst_hbm, out_hbm, ones_v, *rest):
    didx = rest[0:PD]
    acc_sh = rest[PD]
    isem = rest[PD + 1:2 * PD + 1]
    ssem = rest[2 * PD + 1:3 * PD + 1]

    cid = lax.axis_index("c")
    sid = lax.axis_index("s")
    ebase = cid * EPC + sid * EPT

    pltpu.sync_copy(ones_hbm.at[pl.ds(0, CH)], ones_v)
    _init_acc(ones_hbm, acc_sh, sid)
    plsc.subcore_barrier()

    def fire(j, b):
        pltpu.async_copy(dst_hbm.at[pl.ds(ebase + j * CH, CH)], didx[b],
                         isem[b])

    def scat(b):
        pltpu.make_async_copy(dst_hbm.at[pl.ds(ebase, CH)], didx[b],
                              isem[b]).wait()
        pltpu.async_copy(ones_v, acc_sh.at[didx[b]], ssem[b], add=True)

    def scat_wait(b):
        pltpu.make_async_copy(ones_v, acc_sh.at[didx[b]], ssem[b]).wait()

    for b in range(PD):
        fire(b, b)
    for b in range(PD):
        scat(b)

    @pl.loop(1, NG)
    def _(g):
        base = g * PD
        for b in range(PD):
            scat_wait(b)
            fire(base + b, b)
        for b in range(PD):
            scat(b)

    for b in range(PD):
        scat_wait(b)

    plsc.subcore_barrier()
    _write_acc(acc_sh, out_hbm, cid, sid)


@functools.partial(
    pl.kernel,
    out_type=jax.ShapeDtypeStruct((E,), jnp.float32),
    mesh=_mesh,
    scratch_types=(
        [pltpu.VMEM((EPT,), jnp.int32)] * 2
        + [pltpu.VMEM((CH, D), jnp.float32)] * 8
        + [pltpu.VMEM((CH, 16), jnp.float32)]
        + [pltpu.VMEM((CH,), jnp.float32)] * 4
        + [pltpu.VMEM((16,), jnp.float32)]
        + [pltpu.SemaphoreType.DMA] * 8
    ),
    compiler_params=_cp,
)
def _edge_sc(g_hbm, h_hbm, src_hbm, dst_hbm, fcb_hbm, out_hbm,
             sbuf, dbuf, gb0, hb0, gb1, hb1, gb2, hb2, gb3, hb3, accb,
             ob0, ob1, ob2, ob3, fcbv, gs0, gs1, gs2, gs3,
             os0, os1, os2, os3):
    cid = lax.axis_index("c")
    sid = lax.axis_index("s")
    ebase = cid * EPC + sid * EPT

    pltpu.sync_copy(fcb_hbm, fcbv)
    pltpu.sync_copy(src_hbm.at[pl.ds(ebase, EPT)], sbuf)
    pltpu.sync_copy(dst_hbm.at[pl.ds(ebase, EPT)], dbuf)

    gbuf = (gb0, gb1, gb2, gb3)
    hbuf = (hb0, hb1, hb2, hb3)
    obuf = (ob0, ob1, ob2, ob3)
    gsem = (gs0, gs1, gs2, gs3)
    osem = (os0, os1, os2, os3)
    lanes = lax.iota(jnp.int32, 16)

    def fire(j, b):
        pltpu.async_copy(g_hbm.at[sbuf.at[pl.ds(j * CH, CH)]], gbuf[b],
                         gsem[b])
        pltpu.async_copy(h_hbm.at[dbuf.at[pl.ds(j * CH, CH)]], hbuf[b],
                         gsem[b])

    def gwait(b):
        pltpu.make_async_copy(g_hbm.at[sbuf.at[pl.ds(0, CH)]], gbuf[b],
                              gsem[b]).wait()
        pltpu.make_async_copy(h_hbm.at[dbuf.at[pl.ds(0, CH)]], hbuf[b],
                              gsem[b]).wait()

    def owait(b):
        pltpu.make_async_copy(obuf[b], out_hbm.at[pl.ds(ebase, CH)],
                              osem[b]).wait()

    def compute(j, b):
        gb, hb, ob = gbuf[b], hbuf[b], obuf[b]

        @pl.loop(0, CH)
        def _(r):
            acc = gb[r, pl.ds(0, 16)] * hb[r, pl.ds(0, 16)]
            for t in range(1, D // 16):
                acc = acc + gb[r, pl.ds(16 * t, 16)] * hb[r, pl.ds(16 * t, 16)]
            accb[r] = acc

        @pl.loop(0, CH // 16)
        def _(q):
            rows16 = q * 16 + lanes
            s = plsc.load_gather(accb, [rows16, jnp.full((16,), 0, jnp.int32)])
            for c in range(1, 16):
                s = s + plsc.load_gather(accb,
                                         [rows16, jnp.full((16,), c, jnp.int32)])
            v = s + fcbv[...]
            ob[pl.ds(q * 16, 16)] = 1.0 / (1.0 + jnp.exp(-v))

        pltpu.async_copy(ob, out_hbm.at[pl.ds(ebase + j * CH, CH)], osem[b])

    for b in range(4):
        fire(b, b)
    for b in range(4):
        gwait(b)
        compute(b, b)
        fire(b + 4, b)

    @pl.loop(1, (NCH - 1) // 4 - 1)
    def _(t):
        for b in range(4):
            j = 4 * t + b
            gwait(b)
            owait(b)
            compute(j, b)
            fire(j + 4, b)

    for b in range(4):
        gwait(b)
        owait(b)
        compute(NCH - 5 + b, b)
        if b == 0:
            fire(NCH - 1, 0)
    gwait(0)
    owait(0)
    compute(NCH - 1, 0)
    for b in range(4):
        owait(b)


def _tc_pre(degp, x, w0):
    def body(degp_ref, x_ref, w_ref, y_ref, dis_ref):
        deg = degp_ref[0:N, 0:1] + degp_ref[N:2 * N, 0:1] - 1.0
        dis = lax.rsqrt(deg)
        xw = jnp.dot(x_ref[...], w_ref[...],
                     preferred_element_type=jnp.float32,
                     precision=lax.Precision.HIGHEST)
        y_ref[...] = xw * dis
        dis_ref[...] = dis

    return pl.pallas_call(
        body,
        out_shape=[jax.ShapeDtypeStruct((N, D), jnp.float32),
                   jax.ShapeDtypeStruct((N, 1), jnp.float32)],
    )(degp, x, w0)


def _bn_relu(accp_ref, y_ref, dis_ref, b_ref, ga_ref, be_ref):
    t = dis_ref[...] * (accp_ref[0:N] + accp_ref[N:2 * N] - y_ref[...]) + b_ref[...]
    mu = jnp.mean(t, axis=0, keepdims=True)
    var = jnp.mean((t - mu) ** 2, axis=0, keepdims=True)
    return jnp.maximum(
        ga_ref[...] * (t - mu) * lax.rsqrt(var + 1e-5) + be_ref[...], 0.0)


def _tc_mid(accp, y, dis, b, ga, be, wn):
    def body(accp_ref, y_ref, dis_ref, b_ref, ga_ref, be_ref, w_ref, yn_ref):
        hn = _bn_relu(accp_ref, y_ref, dis_ref, b_ref, ga_ref, be_ref)
        yn_ref[...] = jnp.dot(hn, w_ref[...],
                              preferred_element_type=jnp.float32,
                              precision=lax.Precision.HIGHEST) * dis_ref[...]

    return pl.pallas_call(
        body,
        out_shape=jax.ShapeDtypeStruct((N, D), jnp.float32),
    )(accp, y, dis, b, ga, be, wn)


def _tc_last(accp, y, dis, b, ga, be, fcw_row):
    def body(accp_ref, y_ref, dis_ref, b_ref, ga_ref, be_ref, fcw_ref,
             g_ref, h_ref):
        hn = _bn_relu(accp_ref, y_ref, dis_ref, b_ref, ga_ref, be_ref)
        h_ref[...] = hn
        g_ref[...] = hn * fcw_ref[...]

    return pl.pallas_call(
        body,
        out_shape=[jax.ShapeDtypeStruct((N, D), jnp.float32),
                   jax.ShapeDtypeStruct((N, D), jnp.float32)],
    )(accp, y, dis, b, ga, be, fcw_row)


def kernel(x, edge_index, Ws, bs, gammas, betas, fcW, fcb):
    src = edge_index[0]
    dst = edge_index[1]

    degp = _deg_sc(jnp.ones((N, D), jnp.float32), dst)
    y, dis = _tc_pre(degp, x, Ws[0])
    for i in range(L):
        accp = _agg_sc(y, src, dst)
        if i < L - 1:
            y = _tc_mid(accp, y, dis, bs[i][None], gammas[i][None],
                        betas[i][None], Ws[i + 1])
        else:
            g, h5 = _tc_last(accp, y, dis, bs[i][None], gammas[i][None],
                             betas[i][None], fcW[:, 0][None])
    out = _edge_sc(g, h5, src, dst, jnp.broadcast_to(fcb, (16,)))
    return out[:, None]


# speedup vs baseline: 18.7955x; 1.0033x over previous
import dataclasses
import functools

import jax
import jax.numpy as jnp
from jax import lax
from jax.experimental import pallas as pl
from jax.experimental.pallas import tpu as pltpu
from jax.experimental.pallas import tpu_sc as plsc

N = 10000
E = 320000
D = 128
L = 5

NC = 2
NS = 16
RPT = 624
RTAIL = N - NS * RPT
EPC = E // NC
EPT = EPC // NS
CH = 80
NCH = EPT // CH
PD = 5
NG = NCH // PD
CHA = 40
NCHA = EPT // CHA
NGA = NCHA // PD

_mesh = plsc.VectorSubcoreMesh(core_axis_name="c", subcore_axis_name="s")

_cp = pltpu.CompilerParams()
if "needs_layout_passes" in pltpu.CompilerParams.__dataclass_fields__:
    _cp = dataclasses.replace(_cp, needs_layout_passes=False)


def _init_acc(src_hbm, acc_sh, sid):
    rbase = sid * RPT
    pltpu.sync_copy(src_hbm.at[pl.ds(rbase, RPT)], acc_sh.at[pl.ds(rbase, RPT)])

    @pl.when(sid == NS - 1)
    def _():
        pltpu.sync_copy(src_hbm.at[pl.ds(NS * RPT, RTAIL)],
                        acc_sh.at[pl.ds(NS * RPT, RTAIL)])


def _write_acc(acc_sh, out_hbm, cid, sid):
    rbase = sid * RPT
    pltpu.sync_copy(acc_sh.at[pl.ds(rbase, RPT)],
                    out_hbm.at[pl.ds(cid * N + rbase, RPT)])

    @pl.when(sid == NS - 1)
    def _():
        pltpu.sync_copy(acc_sh.at[pl.ds(NS * RPT, RTAIL)],
                        out_hbm.at[pl.ds(cid * N + NS * RPT, RTAIL)])


AP = 4
AGRP = (NCH - 1) // AP


@functools.partial(
    pl.kernel,
    out_type=jax.ShapeDtypeStruct((NC * N, D), jnp.float32),
    mesh=_mesh,
    scratch_types=(
        [pltpu.VMEM((CH,), jnp.int32)] * AP
        + [pltpu.VMEM((CH,), jnp.int32)] * AP
        + [pltpu.VMEM((CH, D), jnp.float32)] * AP
        + [pltpu.VMEM_SHARED((N, D), jnp.float32)]
        + [pltpu.SemaphoreType.DMA] * (4 * AP)
    ),
)
def _agg_sc(y_hbm, src_hbm, dst_hbm, out_hbm, *rest):
    sidx = rest[0:AP]
    didx = rest[AP:2 * AP]
    rows = rest[2 * AP:3 * AP]
    acc_sh = rest[3 * AP]
    ssm = rest[3 * AP + 1:4 * AP + 1]
    ism = rest[4 * AP + 1:5 * AP + 1]
    gsm = rest[5 * AP + 1:6 * AP + 1]
    scm = rest[6 * AP + 1:7 * AP + 1]

    cid = lax.axis_index("c")
    sid = lax.axis_index("s")
    ebase = cid * EPC + sid * EPT

    _init_acc(y_hbm, acc_sh, sid)
    plsc.subcore_barrier()

    def sfire(j, b):
        pltpu.async_copy(src_hbm.at[pl.ds(ebase + j * CH, CH)], sidx[b],
                         ssm[b])

    def dfire(j, b):
        pltpu.async_copy(dst_hbm.at[pl.ds(ebase + j * CH, CH)], didx[b],
                         ism[b])

    def gfire(b):
        pltpu.make_async_copy(src_hbm.at[pl.ds(ebase, CH)], sidx[b],
                              ssm[b]).wait()
        pltpu.async_copy(y_hbm.at[sidx[b]], rows[b], gsm[b])

    def gwait(b):
        pltpu.make_async_copy(y_hbm.at[sidx[b]], rows[b], gsm[b]).wait()

    def scat(b):
        pltpu.make_async_copy(dst_hbm.at[pl.ds(ebase, CH)], didx[b],
                              ism[b]).wait()
        pltpu.async_copy(rows[b], acc_sh.at[didx[b]], scm[b], add=True)

    def scwait(b):
        pltpu.make_async_copy(rows[b], acc_sh.at[didx[b]], scm[b]).wait()

    for b in range(AP):
        sfire(b, b)
        dfire(b, b)
    for b in range(AP):
        gfire(b)
    for b in range(AP):
        gwait(b)
        sfire(AP + b, b)
        scat(b)

    @pl.loop(1, AGRP - 1)
    def _(g):
        for b in range(AP):
            scwait(b)
            dfire(4 * g + b, b)
            gfire(b)
        for b in range(AP):
            gwait(b)
            sfire(4 * (g + 1) + b, b)
            scat(b)

    gl = AGRP - 1
    for b in range(AP):
        scwait(b)
        dfire(4 * gl + b, b)
        gfire(b)
    for b in range(AP):
        gwait(b)
        scat(b)

    scwait(0)
    sfire(NCH - 1, 0)
    dfire(NCH - 1, 0)
    gfire(0)
    gwait(0)
    scat(0)

    for b in range(AP):
        scwait(b)

    plsc.subcore_barrier()
    _write_acc(acc_sh, out_hbm, cid, sid)


@functools.partial(
    pl.kernel,
    out_type=jax.ShapeDtypeStruct((NC * N, D), jnp.float32),
    mesh=_mesh,
    scratch_types=(
        [pltpu.VMEM((CH, D), jnp.float32)]
        + [pltpu.VMEM((CH,), jnp.int32)] * PD
        + [pltpu.VMEM_SHARED((N, D), jnp.float32)]
        + [pltpu.SemaphoreType.DMA] * (2 * PD)
    ),
)
def _deg_sc(ones_hbm, dst_hbm, out_hbm, ones_v, *rest):
    didx = rest[0:PD]
    acc_sh = rest[PD]
    isem = rest[PD + 1:2 * PD + 1]
    ssem = rest[2 * PD + 1:3 * PD + 1]

    cid = lax.axis_index("c")
    sid = lax.axis_index("s")
    ebase = cid * EPC + sid * EPT

    pltpu.sync_copy(ones_hbm.at[pl.ds(0, CH)], ones_v)
    _init_acc(ones_hbm, acc_sh, sid)
    plsc.subcore_barrier()

    def fire(j, b):
        pltpu.async_copy(dst_hbm.at[pl.ds(ebase + j * CH, CH)], didx[b],
                         isem[b])

    def scat(b):
        pltpu.make_async_copy(dst_hbm.at[pl.ds(ebase, CH)], didx[b],
                              isem[b]).wait()
        pltpu.async_copy(ones_v, acc_sh.at[didx[b]], ssem[b], add=True)

    def scat_wait(b):
        pltpu.make_async_copy(ones_v, acc_sh.at[didx[b]], ssem[b]).wait()

    for b in range(PD):
        fire(b, b)
    for b in range(PD):
        scat(b)

    @pl.loop(1, NG)
    def _(g):
        base = g * PD
        for b in range(PD):
            scat_wait(b)
            fire(base + b, b)
        for b in range(PD):
            scat(b)

    for b in range(PD):
        scat_wait(b)

    plsc.subcore_barrier()
    _write_acc(acc_sh, out_hbm, cid, sid)


@functools.partial(
    pl.kernel,
    out_type=jax.ShapeDtypeStruct((E,), jnp.float32),
    mesh=_mesh,
    scratch_types=(
        [pltpu.VMEM((EPT,), jnp.int32)] * 2
        + [pltpu.VMEM((CH, D), jnp.float32)] * 8
        + [pltpu.VMEM((CH, 16), jnp.float32)]
        + [pltpu.VMEM((CH,), jnp.float32)] * 4
        + [pltpu.VMEM((16,), jnp.float32)]
        + [pltpu.SemaphoreType.DMA] * 8
    ),
    compiler_params=_cp,
)
def _edge_sc(g_hbm, h_hbm, src_hbm, dst_hbm, fcb_hbm, out_hbm,
             sbuf, dbuf, gb0, hb0, gb1, hb1, gb2, hb2, gb3, hb3, accb,
             ob0, ob1, ob2, ob3, fcbv, gs0, gs1, gs2, gs3,
             os0, os1, os2, os3):
    cid = lax.axis_index("c")
    sid = lax.axis_index("s")
    ebase = cid * EPC + sid * EPT

    pltpu.sync_copy(fcb_hbm, fcbv)
    pltpu.sync_copy(src_hbm.at[pl.ds(ebase, EPT)], sbuf)
    pltpu.sync_copy(dst_hbm.at[pl.ds(ebase, EPT)], dbuf)

    gbuf = (gb0, gb1, gb2, gb3)
    hbuf = (hb0, hb1, hb2, hb3)
    obuf = (ob0, ob1, ob2, ob3)
    gsem = (gs0, gs1, gs2, gs3)
    osem = (os0, os1, os2, os3)
    lanes = lax.iota(jnp.int32, 16)

    def fire(j, b):
        pltpu.async_copy(g_hbm.at[sbuf.at[pl.ds(j * CH, CH)]], gbuf[b],
                         gsem[b])
        pltpu.async_copy(h_hbm.at[dbuf.at[pl.ds(j * CH, CH)]], hbuf[b],
                         gsem[b])

    def gwait(b):
        pltpu.make_async_copy(g_hbm.at[sbuf.at[pl.ds(0, CH)]], gbuf[b],
                              gsem[b]).wait()
        pltpu.make_async_copy(h_hbm.at[dbuf.at[pl.ds(0, CH)]], hbuf[b],
                              gsem[b]).wait()

    def owait(b):
        pltpu.make_async_copy(obuf[b], out_hbm.at[pl.ds(ebase, CH)],
                              osem[b]).wait()

    def compute(j, b):
        gb, hb, ob = gbuf[b], hbuf[b], obuf[b]

        @pl.loop(0, CH)
        def _(r):
            acc = gb[r, pl.ds(0, 16)] * hb[r, pl.ds(0, 16)]
            for t in range(1, D // 16):
                acc = acc + gb[r, pl.ds(16 * t, 16)] * hb[r, pl.ds(16 * t, 16)]
            accb[r] = acc

        @pl.loop(0, CH // 16)
        def _(q):
            rows16 = q * 16 + lanes
            s = plsc.load_gather(accb, [rows16, jnp.full((16,), 0, jnp.int32)])
            for c in range(1, 16):
                s = s + plsc.load_gather(accb,
                                         [rows16, jnp.full((16,), c, jnp.int32)])
            v = s + fcbv[...]
            ob[pl.ds(q * 16, 16)] = 1.0 / (1.0 + jnp.exp(-v))

        pltpu.async_copy(ob, out_hbm.at[pl.ds(ebase + j * CH, CH)], osem[b])

    for b in range(4):
        fire(b, b)
    for b in range(4):
        gwait(b)
        compute(b, b)
        fire(b + 4, b)

    @pl.loop(1, (NCH - 1) // 4 - 1)
    def _(t):
        for b in range(4):
            j = 4 * t + b
            gwait(b)
            owait(b)
            compute(j, b)
            fire(j + 4, b)

    for b in range(4):
        gwait(b)
        owait(b)
        compute(NCH - 5 + b, b)
        if b == 0:
            fire(NCH - 1, 0)
    gwait(0)
    owait(0)
    compute(NCH - 1, 0)
    for b in range(4):
        owait(b)


def _tc_pre(degp, x, w0):
    def body(degp_ref, x_ref, w_ref, y_ref, dis_ref):
        deg = degp_ref[0:N, 0:1] + degp_ref[N:2 * N, 0:1] - 1.0
        dis = lax.rsqrt(deg)
        xw = jnp.dot(x_ref[...], w_ref[...],
                     preferred_element_type=jnp.float32,
                     precision=lax.Precision.HIGHEST)
        y_ref[...] = xw * dis
        dis_ref[...] = dis

    return pl.pallas_call(
        body,
        out_shape=[jax.ShapeDtypeStruct((N, D), jnp.float32),
                   jax.ShapeDtypeStruct((N, 1), jnp.float32)],
    )(degp, x, w0)


def _bn_relu(accp_ref, y_ref, dis_ref, b_ref, ga_ref, be_ref):
    t = dis_ref[...] * (accp_ref[0:N] + accp_ref[N:2 * N] - y_ref[...]) + b_ref[...]
    mu = jnp.mean(t, axis=0, keepdims=True)
    var = jnp.mean((t - mu) ** 2, axis=0, keepdims=True)
    return jnp.maximum(
        ga_ref[...] * (t - mu) * lax.rsqrt(var + 1e-5) + be_ref[...], 0.0)


def _tc_mid(accp, y, dis, b, ga, be, wn):
    def body(accp_ref, y_ref, dis_ref, b_ref, ga_ref, be_ref, w_ref, yn_ref):
        hn = _bn_relu(accp_ref, y_ref, dis_ref, b_ref, ga_ref, be_ref)
        yn_ref[...] = jnp.dot(hn, w_ref[...],
                              preferred_element_type=jnp.float32,
                              precision=lax.Precision.HIGHEST) * dis_ref[...]

    return pl.pallas_call(
        body,
        out_shape=jax.ShapeDtypeStruct((N, D), jnp.float32),
    )(accp, y, dis, b, ga, be, wn)


def _tc_last(accp, y, dis, b, ga, be, fcw_row):
    def body(accp_ref, y_ref, dis_ref, b_ref, ga_ref, be_ref, fcw_ref,
             g_ref, h_ref):
        hn = _bn_relu(accp_ref, y_ref, dis_ref, b_ref, ga_ref, be_ref)
        h_ref[...] = hn
        g_ref[...] = hn * fcw_ref[...]

    return pl.pallas_call(
        body,
        out_shape=[jax.ShapeDtypeStruct((N, D), jnp.float32),
                   jax.ShapeDtypeStruct((N, D), jnp.float32)],
    )(accp, y, dis, b, ga, be, fcw_row)


def kernel(x, edge_index, Ws, bs, gammas, betas, fcW, fcb):
    src = edge_index[0]
    dst = edge_index[1]

    degp = _deg_sc(jnp.ones((N, D), jnp.float32), dst)
    y, dis = _tc_pre(degp, x, Ws[0])
    for i in range(L):
        accp = _agg_sc(y, src, dst)
        if i < L - 1:
            y = _tc_mid(accp, y, dis, bs[i][None], gammas[i][None],
                        betas[i][None], Ws[i + 1])
        else:
            g, h5 = _tc_last(accp, y, dis, bs[i][None], gammas[i][None],
                             betas[i][None], fcW[:, 0][None])
    out = _edge_sc(g, h5, src, dst, jnp.broadcast_to(fcb, (16,)))
    return out[:, None]
